# baseline (speedup 1.0000x reference)
import functools
import math

import jax
import jax.numpy as jnp
from jax import lax
from jax.experimental import pallas as pl
from jax.experimental.pallas import tpu as pltpu
from jax.experimental.pallas import tpu_sc as plsc


def _build_counts_sc(idx, l, s):
    n_chunks = 4
    chunk_rows = l // n_chunks
    rows_per_tile = chunk_rows // 16
    tile_elems = rows_per_tile * l
    zlen = 8192
    mesh = plsc.VectorSubcoreMesh(core_axis_name="c", subcore_axis_name="s")

    @functools.partial(
        pl.kernel, mesh=mesh,
        out_type=jax.ShapeDtypeStruct((l * l,), jnp.float32),
        scratch_types=[
            pltpu.VMEM((rows_per_tile, s), jnp.int32),
            pltpu.VMEM((s // 128, 128), jnp.int32),
            pltpu.VMEM((128,), jnp.float32),
            pltpu.VMEM((zlen,), jnp.float32),
            pltpu.VMEM_SHARED((chunk_rows * l,), jnp.float32),
        ],
    )
    def build(idx_hbm, c_hbm, idx_v, abs_v, ones_v, zeros_v, chunk_sh):
        cid = lax.axis_index("c")
        sid = lax.axis_index("s")
        for i in range(128 // 16):
            ones_v[pl.ds(i * 16, 16)] = jnp.ones((16,), jnp.float32)
        for i in range(zlen // 16):
            zeros_v[pl.ds(i * 16, 16)] = jnp.zeros((16,), jnp.float32)
        my_base = sid * tile_elems
        for r in range(2):
            chunk_id = cid * 2 + r
            for zi in range(tile_elems // zlen):
                pltpu.sync_copy(zeros_v,
                                chunk_sh.at[pl.ds(my_base + zi * zlen, zlen)])
            row0 = chunk_id * chunk_rows + sid * rows_per_tile
            pltpu.sync_copy(idx_hbm.at[pl.ds(row0, rows_per_tile)], idx_v)

            def row_body(rr, carry):
                base = (sid * rows_per_tile + rr) * l
                for i in range(s // 16):
                    vec = idx_v[rr, pl.ds(i * 16, 16)] + base
                    abs_v[i // 8, pl.ds((i % 8) * 16, 16)] = vec
                for j in range(s // 128):
                    pltpu.sync_copy(ones_v, chunk_sh.at[abs_v.at[j]],
                                    add=True)
                return carry

            lax.fori_loop(0, rows_per_tile, row_body, 0)
            pltpu.sync_copy(
                chunk_sh.at[pl.ds(my_base, tile_elems)],
                c_hbm.at[pl.ds(chunk_id * chunk_rows * l + my_base,
                               tile_elems)])

    return build(idx)


def _score_kernel(c_ref, q_ref, k_ref, m_ref, *, nheads, hdim, l_key):
    b = pl.program_id(1)
    c = c_ref[...]
    pen = jnp.where(c > 0.0, 0.0, -jnp.inf)
    q = q_ref[0]
    k = k_ref[b]
    for h in range(nheads):
        qh = q[:, h * hdim:(h + 1) * hdim]
        kh = k[:, h * hdim:(h + 1) * hdim]
        sc = jax.lax.dot_general(qh, kh, (((1,), (1,)), ((), ())),
                                 preferred_element_type=jnp.float32)
        ssum = jnp.sum(sc * c, axis=1)
        smax = jnp.max(sc + pen, axis=1)
        m_ref[0, 0, h, :] = smax - ssum * (1.0 / l_key)


def _sortable_u32(x):
    bits = jax.lax.bitcast_convert_type(x, jnp.uint32)
    flip = jnp.where(bits >> 31 == jnp.uint32(1),
                     jnp.uint32(0xFFFFFFFF), jnp.uint32(0x80000000))
    return bits ^ flip


def _threshold_kernel(m_ref, t_ref, *, nq):
    us = _sortable_u32(m_ref[...])
    rows = us.shape[0]

    def body(_, lohi):
        lo, hi = lohi
        mid = hi - ((hi - lo) >> 1)
        cnt = jnp.sum(jnp.where(us >= mid, 1.0, 0.0), axis=1, keepdims=True)
        ge = cnt >= nq
        return jnp.where(ge, mid, lo), jnp.where(ge, hi, mid - 1)

    t, _ = jax.lax.fori_loop(
        0, 32, body,
        (jnp.zeros((rows, 1), jnp.uint32),
         jnp.full((rows, 1), 0xFFFFFFFF, jnp.uint32)))
    t_ref[...] = t


def _excl_cumsum_rows(x, rows):
    n, l = x.shape
    cols = l // rows
    nn = n * rows
    xs = jnp.concatenate([x[:, i * cols:(i + 1) * cols] for i in range(rows)],
                         axis=0)
    iu = jax.lax.broadcasted_iota(jnp.int32, (cols, cols), 0)
    ju = jax.lax.broadcasted_iota(jnp.int32, (cols, cols), 1)
    tu = jnp.where(iu < ju, 1.0, 0.0)
    ex = jax.lax.dot_general(xs, tu, (((1,), (0,)), ((), ())),
                             preferred_element_type=jnp.float32)
    rs = jnp.sum(xs, axis=1, keepdims=True)
    il = jax.lax.broadcasted_iota(jnp.int32, (nn, nn), 0)
    jl = jax.lax.broadcasted_iota(jnp.int32, (nn, nn), 1)
    tl = jnp.where((jl % n == il % n) & (jl // n < il // n), 1.0, 0.0)
    off = jax.lax.dot_general(tl, rs, (((1,), (0,)), ((), ())),
                              preferred_element_type=jnp.float32)
    r = ex + off
    return jnp.concatenate(
        [jnp.concatenate([r[i * n + rr:i * n + rr + 1, :]
                          for i in range(rows)], axis=1)
         for rr in range(n)], axis=0)


def _attn_kernel(m_ref, t_ref, q_ref, k_ref, v_ref, o_ref, *,
                 nq, scale, srows, hpg, hdim):
    l = m_ref.shape[3]
    for hh in range(hpg):
        us = _sortable_u32(m_ref[0, hh])
        t = t_ref[0, hh]
        gt = us > t
        eq = us == t
        n_gt = jnp.sum(jnp.where(gt, 1.0, 0.0))
        need = nq - n_gt
        both = jnp.concatenate(
            [jnp.where(gt, 1.0, 0.0), jnp.where(eq, 1.0, 0.0)], axis=0)
        cums = _excl_cumsum_rows(both, srows)
        cum_gt = cums[0:1]
        cum_eq = cums[1:2]
        sel = jnp.logical_or(gt, jnp.logical_and(eq, cum_eq < need))
        rank = cum_gt + jnp.minimum(cum_eq, need)
        riota = jax.lax.broadcasted_iota(
            jnp.int32, (nq, l), 0).astype(jnp.float32)
        p = jnp.where((riota == rank) & sel, 1.0, 0.0)

        q = q_ref[0][:, hh * hdim:(hh + 1) * hdim]
        k = k_ref[0][:, hh * hdim:(hh + 1) * hdim]
        v = v_ref[0][:, hh * hdim:(hh + 1) * hdim]
        q_sel = jax.lax.dot_general(p, q, (((1,), (0,)), ((), ())),
                                    preferred_element_type=jnp.float32)
        sim = jax.lax.dot_general(q_sel, k, (((1,), (1,)), ((), ())),
                                  preferred_element_type=jnp.float32) * scale
        sim_max = jnp.max(sim, axis=1, keepdims=True)
        e = jnp.exp(sim - sim_max)
        a = e / jnp.sum(e, axis=1, keepdims=True)
        attn = jax.lax.dot_general(a, v, (((1,), (0,)), ((), ())),
                                   preferred_element_type=jnp.float32)
        scat = jax.lax.dot_general(p, attn, (((0,), (0,)), ((), ())),
                                   preferred_element_type=jnp.float32)
        ones_col = jnp.ones((nq, 1), jnp.float32)
        selcol = jax.lax.dot_general(p, ones_col, (((0,), (0,)), ((), ())),
                                     preferred_element_type=jnp.float32)
        meanv = jnp.sum(v, axis=0, keepdims=True) * (1.0 / l)
        o_ref[0, :, hh * hdim:(hh + 1) * hdim] = scat + (1.0 - selcol) * meanv


def kernel(query, key, value, index_key):
    b, l, h, d = query.shape
    hd = h * d
    nq = l // 4
    bh = b * h
    scale = 1.0 / math.sqrt(d)
    q2 = query.reshape(b, l, hd)
    k2 = key.reshape(b, l, hd)
    v2 = value.reshape(b, l, hd)
    idx = index_key.astype(jnp.int32)
    s_samp = idx.shape[1]
    if l % 64 == 0 and s_samp % 128 == 0:
        c = _build_counts_sc(idx, l, s_samp).reshape(l, l)
    else:
        rows = jnp.arange(l, dtype=jnp.int32)[:, None]
        c = jnp.zeros((l, l), jnp.float32).at[rows, idx].add(1.0)

    qt = min(512, l)
    nqt = l // qt
    m = pl.pallas_call(
        functools.partial(_score_kernel, nheads=h, hdim=d, l_key=l),
        grid=(nqt, b),
        in_specs=[
            pl.BlockSpec((qt, l), lambda i, j: (i, 0)),
            pl.BlockSpec((1, qt, hd), lambda i, j: (j, i, 0)),
            pl.BlockSpec((b, l, hd), lambda i, j: (0, 0, 0)),
        ],
        out_specs=pl.BlockSpec((1, 1, h, qt), lambda i, j: (j, i, 0, 0)),
        out_shape=jax.ShapeDtypeStruct((b, nqt, h, qt), jnp.float32),
    )(c, q2, k2)
    m_flat = jnp.transpose(m, (0, 2, 1, 3)).reshape(bh, l)

    t = pl.pallas_call(
        functools.partial(_threshold_kernel, nq=nq),
        grid=(1,),
        in_specs=[pl.BlockSpec((bh, l), lambda i: (0, 0))],
        out_specs=pl.BlockSpec((bh, 1), lambda i: (0, 0)),
        out_shape=jax.ShapeDtypeStruct((bh, 1), jnp.uint32),
    )(m_flat)

    hpg = 8 if h % 8 == 0 else (4 if h % 4 == 0 else (2 if h % 2 == 0 else 1))
    nhg = h // hpg
    srows = 16 if l % 16 == 0 else 8
    m4 = m_flat.reshape(b, h, 1, l)
    t4 = t.reshape(b, h, 1, 1)
    out = pl.pallas_call(
        functools.partial(_attn_kernel, nq=nq, scale=scale, srows=srows,
                          hpg=hpg, hdim=d),
        grid=(b, nhg),
        in_specs=[
            pl.BlockSpec((1, hpg, 1, l), lambda i, j: (i, j, 0, 0)),
            pl.BlockSpec((1, hpg, 1, 1), lambda i, j: (i, j, 0, 0)),
            pl.BlockSpec((1, l, hpg * d), lambda i, j: (i, 0, j)),
            pl.BlockSpec((1, l, hpg * d), lambda i, j: (i, 0, j)),
            pl.BlockSpec((1, l, hpg * d), lambda i, j: (i, 0, j)),
        ],
        out_specs=pl.BlockSpec((1, l, hpg * d), lambda i, j: (i, 0, j)),
        out_shape=jax.ShapeDtypeStruct((b, l, hd), jnp.float32),
    )(m4, t4, q2, k2, v2)
    return out.reshape(b, l, h, d)


# speedup vs baseline: 1.0369x; 1.0369x over previous
import functools
import math

import jax
import jax.numpy as jnp
from jax import lax
from jax.experimental import pallas as pl
from jax.experimental.pallas import tpu as pltpu
from jax.experimental.pallas import tpu_sc as plsc


def _build_counts_sc(idx, l, s):
    n_chunks = 4
    chunk_rows = l // n_chunks
    rows_per_tile = chunk_rows // 16
    tile_elems = rows_per_tile * l
    zlen = 8192
    mesh = plsc.VectorSubcoreMesh(core_axis_name="c", subcore_axis_name="s")

    @functools.partial(
        pl.kernel, mesh=mesh,
        out_type=jax.ShapeDtypeStruct((l * l,), jnp.float32),
        scratch_types=[
            pltpu.VMEM((rows_per_tile, s), jnp.int32),
            pltpu.VMEM((s // 128, 128), jnp.int32),
            pltpu.VMEM((128,), jnp.float32),
            pltpu.VMEM((zlen,), jnp.float32),
            pltpu.VMEM_SHARED((chunk_rows * l,), jnp.float32),
        ],
    )
    def build(idx_hbm, c_hbm, idx_v, abs_v, ones_v, zeros_v, chunk_sh):
        cid = lax.axis_index("c")
        sid = lax.axis_index("s")
        for i in range(128 // 16):
            ones_v[pl.ds(i * 16, 16)] = jnp.ones((16,), jnp.float32)
        for i in range(zlen // 16):
            zeros_v[pl.ds(i * 16, 16)] = jnp.zeros((16,), jnp.float32)
        my_base = sid * tile_elems
        for r in range(2):
            chunk_id = cid * 2 + r
            for zi in range(tile_elems // zlen):
                pltpu.sync_copy(zeros_v,
                                chunk_sh.at[pl.ds(my_base + zi * zlen, zlen)])
            row0 = chunk_id * chunk_rows + sid * rows_per_tile
            pltpu.sync_copy(idx_hbm.at[pl.ds(row0, rows_per_tile)], idx_v)

            def row_body(rr, carry):
                base = (sid * rows_per_tile + rr) * l
                for i in range(s // 16):
                    vec = idx_v[rr, pl.ds(i * 16, 16)] + base
                    abs_v[i // 8, pl.ds((i % 8) * 16, 16)] = vec
                for j in range(s // 128):
                    pltpu.sync_copy(ones_v, chunk_sh.at[abs_v.at[j]],
                                    add=True)
                return carry

            lax.fori_loop(0, rows_per_tile, row_body, 0)
            pltpu.sync_copy(
                chunk_sh.at[pl.ds(my_base, tile_elems)],
                c_hbm.at[pl.ds(chunk_id * chunk_rows * l + my_base,
                               tile_elems)])

    return build(idx)


def _score_kernel(c_ref, q_ref, k_ref, m_ref, *, nheads, hdim, l_key):
    b = pl.program_id(1)
    c = c_ref[...]
    pen = jnp.where(c > 0.0, 0.0, -jnp.inf)
    q = q_ref[0]
    k = k_ref[b]
    for h in range(nheads):
        qh = q[:, h * hdim:(h + 1) * hdim]
        kh = k[:, h * hdim:(h + 1) * hdim]
        sc = jax.lax.dot_general(qh, kh, (((1,), (1,)), ((), ())),
                                 preferred_element_type=jnp.float32)
        ssum = jnp.sum(sc * c, axis=1)
        smax = jnp.max(sc + pen, axis=1)
        m_ref[0, 0, h, :] = smax - ssum * (1.0 / l_key)


def _sortable_u32(x):
    bits = jax.lax.bitcast_convert_type(x, jnp.uint32)
    flip = jnp.where(bits >> 31 == jnp.uint32(1),
                     jnp.uint32(0xFFFFFFFF), jnp.uint32(0x80000000))
    return bits ^ flip


def _threshold_kernel(m_ref, t_ref, *, nq):
    us = _sortable_u32(m_ref[...])
    rows = us.shape[0]

    def body(_, lohi):
        lo, hi = lohi
        mid = hi - ((hi - lo) >> 1)
        cnt = jnp.sum(jnp.where(us >= mid, 1.0, 0.0), axis=1, keepdims=True)
        ge = cnt >= nq
        return jnp.where(ge, mid, lo), jnp.where(ge, hi, mid - 1)

    t, _ = jax.lax.fori_loop(
        0, 32, body,
        (jnp.zeros((rows, 1), jnp.uint32),
         jnp.full((rows, 1), 0xFFFFFFFF, jnp.uint32)))
    t_ref[...] = t


def _excl_cumsum_rows(x, rows):
    n, l = x.shape
    cols = l // rows
    nn = n * rows
    xs = jnp.concatenate([x[:, i * cols:(i + 1) * cols] for i in range(rows)],
                         axis=0)
    iu = jax.lax.broadcasted_iota(jnp.int32, (cols, cols), 0)
    ju = jax.lax.broadcasted_iota(jnp.int32, (cols, cols), 1)
    tu = jnp.where(iu < ju, 1.0, 0.0)
    ex = jax.lax.dot_general(xs, tu, (((1,), (0,)), ((), ())),
                             preferred_element_type=jnp.float32)
    rs = jnp.sum(xs, axis=1, keepdims=True)
    il = jax.lax.broadcasted_iota(jnp.int32, (nn, nn), 0)
    jl = jax.lax.broadcasted_iota(jnp.int32, (nn, nn), 1)
    tl = jnp.where((jl % n == il % n) & (jl // n < il // n), 1.0, 0.0)
    off = jax.lax.dot_general(tl, rs, (((1,), (0,)), ((), ())),
                              preferred_element_type=jnp.float32)
    r = ex + off
    return jnp.concatenate(
        [jnp.concatenate([r[i * n + rr:i * n + rr + 1, :]
                          for i in range(rows)], axis=1)
         for rr in range(n)], axis=0)


def _attn_kernel(m_ref, t_ref, q_ref, k_ref, v_ref, o_ref, *,
                 nq, scale, srows, hpg, hdim):
    l = m_ref.shape[3]
    for hh in range(hpg):
        us = _sortable_u32(m_ref[0, hh])
        t = t_ref[0, hh]
        gt = us > t
        eq = us == t
        n_gt = jnp.sum(jnp.where(gt, 1.0, 0.0))
        need = nq - n_gt
        both = jnp.concatenate(
            [jnp.where(gt, 1.0, 0.0), jnp.where(eq, 1.0, 0.0)], axis=0)
        cums = _excl_cumsum_rows(both, srows)
        cum_gt = cums[0:1]
        cum_eq = cums[1:2]
        sel = jnp.logical_or(gt, jnp.logical_and(eq, cum_eq < need))
        rank = cum_gt + jnp.minimum(cum_eq, need)
        riota = jax.lax.broadcasted_iota(
            jnp.int32, (nq, l), 0).astype(jnp.float32)
        p = jnp.where((riota == rank) & sel, 1.0, 0.0)

        q = q_ref[0][:, hh * hdim:(hh + 1) * hdim]
        k = k_ref[0][:, hh * hdim:(hh + 1) * hdim]
        v = v_ref[0][:, hh * hdim:(hh + 1) * hdim]
        q_sel = jax.lax.dot_general(p, q, (((1,), (0,)), ((), ())),
                                    preferred_element_type=jnp.float32)
        sim = jax.lax.dot_general(q_sel, k, (((1,), (1,)), ((), ())),
                                  preferred_element_type=jnp.float32) * scale
        sim_max = jnp.max(sim, axis=1, keepdims=True)
        e = jnp.exp(sim - sim_max)
        a = e / jnp.sum(e, axis=1, keepdims=True)
        attn = jax.lax.dot_general(a, v, (((1,), (0,)), ((), ())),
                                   preferred_element_type=jnp.float32)
        scat = jax.lax.dot_general(p, attn, (((0,), (0,)), ((), ())),
                                   preferred_element_type=jnp.float32)
        ones_col = jnp.ones((nq, 1), jnp.float32)
        selcol = jax.lax.dot_general(p, ones_col, (((0,), (0,)), ((), ())),
                                     preferred_element_type=jnp.float32)
        meanv = jnp.sum(v, axis=0, keepdims=True) * (1.0 / l)
        o_ref[0, :, hh * hdim:(hh + 1) * hdim] = scat + (1.0 - selcol) * meanv


def kernel(query, key, value, index_key):
    b, l, h, d = query.shape
    hd = h * d
    nq = l // 4
    bh = b * h
    scale = 1.0 / math.sqrt(d)
    q2 = query.reshape(b, l, hd)
    k2 = key.reshape(b, l, hd)
    v2 = value.reshape(b, l, hd)
    idx = index_key.astype(jnp.int32)
    s_samp = idx.shape[1]
    if l % 64 == 0 and s_samp % 128 == 0:
        c = _build_counts_sc(idx, l, s_samp).reshape(l, l)
    else:
        rows = jnp.arange(l, dtype=jnp.int32)[:, None]
        c = jnp.zeros((l, l), jnp.float32).at[rows, idx].add(1.0)

    qt = min(256, l)
    nqt = l // qt
    m = pl.pallas_call(
        functools.partial(_score_kernel, nheads=h, hdim=d, l_key=l),
        grid=(nqt, b),
        in_specs=[
            pl.BlockSpec((qt, l), lambda i, j: (i, 0)),
            pl.BlockSpec((1, qt, hd), lambda i, j: (j, i, 0)),
            pl.BlockSpec((b, l, hd), lambda i, j: (0, 0, 0)),
        ],
        out_specs=pl.BlockSpec((1, 1, h, qt), lambda i, j: (j, i, 0, 0)),
        out_shape=jax.ShapeDtypeStruct((b, nqt, h, qt), jnp.float32),
    )(c, q2, k2)
    m_flat = jnp.transpose(m, (0, 2, 1, 3)).reshape(bh, l)

    t = pl.pallas_call(
        functools.partial(_threshold_kernel, nq=nq),
        grid=(1,),
        in_specs=[pl.BlockSpec((bh, l), lambda i: (0, 0))],
        out_specs=pl.BlockSpec((bh, 1), lambda i: (0, 0)),
        out_shape=jax.ShapeDtypeStruct((bh, 1), jnp.uint32),
    )(m_flat)

    hpg = 2 if h % 2 == 0 else 1
    nhg = h // hpg
    srows = 16 if l % 16 == 0 else 8
    m4 = m_flat.reshape(b, h, 1, l)
    t4 = t.reshape(b, h, 1, 1)
    out = pl.pallas_call(
        functools.partial(_attn_kernel, nq=nq, scale=scale, srows=srows,
                          hpg=hpg, hdim=d),
        grid=(b, nhg),
        in_specs=[
            pl.BlockSpec((1, hpg, 1, l), lambda i, j: (i, j, 0, 0)),
            pl.BlockSpec((1, hpg, 1, 1), lambda i, j: (i, j, 0, 0)),
            pl.BlockSpec((1, l, hpg * d), lambda i, j: (i, 0, j)),
            pl.BlockSpec((1, l, hpg * d), lambda i, j: (i, 0, j)),
            pl.BlockSpec((1, l, hpg * d), lambda i, j: (i, 0, j)),
        ],
        out_specs=pl.BlockSpec((1, l, hpg * d), lambda i, j: (i, 0, j)),
        out_shape=jax.ShapeDtypeStruct((b, l, hd), jnp.float32),
    )(m4, t4, q2, k2, v2)
    return out.reshape(b, l, h, d)


# speedup vs baseline: 1.0693x; 1.0313x over previous
import functools
import math

import jax
import jax.numpy as jnp
from jax import lax
from jax.experimental import pallas as pl
from jax.experimental.pallas import tpu as pltpu
from jax.experimental.pallas import tpu_sc as plsc


def _build_counts_sc(idx, l, s):
    n_chunks = 4
    chunk_rows = l // n_chunks
    rows_per_tile = chunk_rows // 16
    tile_elems = rows_per_tile * l
    zlen = 8192
    mesh = plsc.VectorSubcoreMesh(core_axis_name="c", subcore_axis_name="s")

    @functools.partial(
        pl.kernel, mesh=mesh,
        out_type=jax.ShapeDtypeStruct((l * l,), jnp.float32),
        scratch_types=[
            pltpu.VMEM((rows_per_tile, s), jnp.int32),
            pltpu.VMEM((s // 128, 128), jnp.int32),
            pltpu.VMEM((128,), jnp.float32),
            pltpu.VMEM((zlen,), jnp.float32),
            pltpu.VMEM_SHARED((chunk_rows * l,), jnp.float32),
            pltpu.SemaphoreType.DMA,
        ],
    )
    def build(idx_hbm, c_hbm, idx_v, abs_v, ones_v, zeros_v, chunk_sh, sem):
        cid = lax.axis_index("c")
        sid = lax.axis_index("s")
        for i in range(128 // 16):
            ones_v[pl.ds(i * 16, 16)] = jnp.ones((16,), jnp.float32)
        for i in range(zlen // 16):
            zeros_v[pl.ds(i * 16, 16)] = jnp.zeros((16,), jnp.float32)
        my_base = sid * tile_elems
        for r in range(2):
            chunk_id = cid * 2 + r
            zcps = [
                pltpu.async_copy(
                    zeros_v, chunk_sh.at[pl.ds(my_base + zi * zlen, zlen)],
                    sem)
                for zi in range(tile_elems // zlen)]
            row0 = chunk_id * chunk_rows + sid * rows_per_tile
            pltpu.sync_copy(idx_hbm.at[pl.ds(row0, rows_per_tile)], idx_v)
            for cp in zcps:
                cp.wait()

            def row_body(rr, carry):
                base = (sid * rows_per_tile + rr) * l
                for i in range(s // 16):
                    vec = idx_v[rr, pl.ds(i * 16, 16)] + base
                    abs_v[i // 8, pl.ds((i % 8) * 16, 16)] = vec
                cps = [pltpu.async_copy(ones_v, chunk_sh.at[abs_v.at[j]],
                                        sem, add=True)
                       for j in range(s // 128)]
                for cp in cps:
                    cp.wait()
                return carry

            lax.fori_loop(0, rows_per_tile, row_body, 0)
            pltpu.sync_copy(
                chunk_sh.at[pl.ds(my_base, tile_elems)],
                c_hbm.at[pl.ds(chunk_id * chunk_rows * l + my_base,
                               tile_elems)])

    return build(idx)


def _score_kernel(c_ref, q_ref, k_ref, m_ref, *, nheads, hdim, l_key):
    b = pl.program_id(1)
    c = c_ref[...]
    pen = jnp.where(c > 0.0, 0.0, -jnp.inf)
    q = q_ref[0]
    k = k_ref[b]
    for h in range(nheads):
        qh = q[:, h * hdim:(h + 1) * hdim]
        kh = k[:, h * hdim:(h + 1) * hdim]
        sc = jax.lax.dot_general(qh, kh, (((1,), (1,)), ((), ())),
                                 preferred_element_type=jnp.float32)
        ssum = jnp.sum(sc * c, axis=1)
        smax = jnp.max(sc + pen, axis=1)
        m_ref[0, 0, h, :] = smax - ssum * (1.0 / l_key)


def _sortable_u32(x):
    bits = jax.lax.bitcast_convert_type(x, jnp.uint32)
    flip = jnp.where(bits >> 31 == jnp.uint32(1),
                     jnp.uint32(0xFFFFFFFF), jnp.uint32(0x80000000))
    return bits ^ flip


def _threshold_kernel(m_ref, t_ref, *, nq):
    us = _sortable_u32(m_ref[...])
    rows = us.shape[0]

    def body(_, lohi):
        lo, hi = lohi
        mid = hi - ((hi - lo) >> 1)
        cnt = jnp.sum(jnp.where(us >= mid, 1.0, 0.0), axis=1, keepdims=True)
        ge = cnt >= nq
        return jnp.where(ge, mid, lo), jnp.where(ge, hi, mid - 1)

    t, _ = jax.lax.fori_loop(
        0, 32, body,
        (jnp.zeros((rows, 1), jnp.uint32),
         jnp.full((rows, 1), 0xFFFFFFFF, jnp.uint32)))
    t_ref[...] = t


def _excl_cumsum_rows(x, rows):
    n, l = x.shape
    cols = l // rows
    nn = n * rows
    xs = jnp.concatenate([x[:, i * cols:(i + 1) * cols] for i in range(rows)],
                         axis=0)
    iu = jax.lax.broadcasted_iota(jnp.int32, (cols, cols), 0)
    ju = jax.lax.broadcasted_iota(jnp.int32, (cols, cols), 1)
    tu = jnp.where(iu < ju, 1.0, 0.0)
    ex = jax.lax.dot_general(xs, tu, (((1,), (0,)), ((), ())),
                             preferred_element_type=jnp.float32)
    rs = jnp.sum(xs, axis=1, keepdims=True)
    il = jax.lax.broadcasted_iota(jnp.int32, (nn, nn), 0)
    jl = jax.lax.broadcasted_iota(jnp.int32, (nn, nn), 1)
    tl = jnp.where((jl % n == il % n) & (jl // n < il // n), 1.0, 0.0)
    off = jax.lax.dot_general(tl, rs, (((1,), (0,)), ((), ())),
                              preferred_element_type=jnp.float32)
    r = ex + off
    return jnp.concatenate(
        [jnp.concatenate([r[i * n + rr:i * n + rr + 1, :]
                          for i in range(rows)], axis=1)
         for rr in range(n)], axis=0)


def _attn_kernel(m_ref, t_ref, q_ref, k_ref, v_ref, o_ref, *,
                 nq, scale, srows, hpg, hdim):
    l = m_ref.shape[3]
    for hh in range(hpg):
        us = _sortable_u32(m_ref[0, hh])
        t = t_ref[0, hh]
        gt = us > t
        eq = us == t
        n_gt = jnp.sum(jnp.where(gt, 1.0, 0.0))
        need = nq - n_gt
        both = jnp.concatenate(
            [jnp.where(gt, 1.0, 0.0), jnp.where(eq, 1.0, 0.0)], axis=0)
        cums = _excl_cumsum_rows(both, srows)
        cum_gt = cums[0:1]
        cum_eq = cums[1:2]
        sel = jnp.logical_or(gt, jnp.logical_and(eq, cum_eq < need))
        rank = cum_gt + jnp.minimum(cum_eq, need)
        riota = jax.lax.broadcasted_iota(
            jnp.int32, (nq, l), 0).astype(jnp.float32)
        p = jnp.where((riota == rank) & sel, 1.0, 0.0)

        q = q_ref[0][:, hh * hdim:(hh + 1) * hdim]
        k = k_ref[0][:, hh * hdim:(hh + 1) * hdim]
        v = v_ref[0][:, hh * hdim:(hh + 1) * hdim]
        q_sel = jax.lax.dot_general(p, q, (((1,), (0,)), ((), ())),
                                    preferred_element_type=jnp.float32)
        sim = jax.lax.dot_general(q_sel, k, (((1,), (1,)), ((), ())),
                                  preferred_element_type=jnp.float32) * scale
        sim_max = jnp.max(sim, axis=1, keepdims=True)
        e = jnp.exp(sim - sim_max)
        a = e / jnp.sum(e, axis=1, keepdims=True)
        attn = jax.lax.dot_general(a, v, (((1,), (0,)), ((), ())),
                                   preferred_element_type=jnp.float32)
        scat = jax.lax.dot_general(p, attn, (((0,), (0,)), ((), ())),
                                   preferred_element_type=jnp.float32)
        ones_col = jnp.ones((nq, 1), jnp.float32)
        selcol = jax.lax.dot_general(p, ones_col, (((0,), (0,)), ((), ())),
                                     preferred_element_type=jnp.float32)
        meanv = jnp.sum(v, axis=0, keepdims=True) * (1.0 / l)
        o_ref[0, :, hh * hdim:(hh + 1) * hdim] = scat + (1.0 - selcol) * meanv


def kernel(query, key, value, index_key):
    b, l, h, d = query.shape
    hd = h * d
    nq = l // 4
    bh = b * h
    scale = 1.0 / math.sqrt(d)
    q2 = query.reshape(b, l, hd)
    k2 = key.reshape(b, l, hd)
    v2 = value.reshape(b, l, hd)
    idx = index_key.astype(jnp.int32)
    s_samp = idx.shape[1]
    if l % 64 == 0 and s_samp % 128 == 0:
        c = _build_counts_sc(idx, l, s_samp).reshape(l, l)
    else:
        rows = jnp.arange(l, dtype=jnp.int32)[:, None]
        c = jnp.zeros((l, l), jnp.float32).at[rows, idx].add(1.0)

    qt = min(256, l)
    nqt = l // qt
    m = pl.pallas_call(
        functools.partial(_score_kernel, nheads=h, hdim=d, l_key=l),
        grid=(nqt, b),
        in_specs=[
            pl.BlockSpec((qt, l), lambda i, j: (i, 0)),
            pl.BlockSpec((1, qt, hd), lambda i, j: (j, i, 0)),
            pl.BlockSpec((b, l, hd), lambda i, j: (0, 0, 0)),
        ],
        out_specs=pl.BlockSpec((1, 1, h, qt), lambda i, j: (j, i, 0, 0)),
        out_shape=jax.ShapeDtypeStruct((b, nqt, h, qt), jnp.float32),
    )(c, q2, k2)
    m_flat = jnp.transpose(m, (0, 2, 1, 3)).reshape(bh, l)

    t = pl.pallas_call(
        functools.partial(_threshold_kernel, nq=nq),
        grid=(1,),
        in_specs=[pl.BlockSpec((bh, l), lambda i: (0, 0))],
        out_specs=pl.BlockSpec((bh, 1), lambda i: (0, 0)),
        out_shape=jax.ShapeDtypeStruct((bh, 1), jnp.uint32),
    )(m_flat)

    hpg = 4 if h % 4 == 0 else (2 if h % 2 == 0 else 1)
    nhg = h // hpg
    srows = 16 if l % 16 == 0 else 8
    m4 = m_flat.reshape(b, h, 1, l)
    t4 = t.reshape(b, h, 1, 1)
    out = pl.pallas_call(
        functools.partial(_attn_kernel, nq=nq, scale=scale, srows=srows,
                          hpg=hpg, hdim=d),
        grid=(b, nhg),
        in_specs=[
            pl.BlockSpec((1, hpg, 1, l), lambda i, j: (i, j, 0, 0)),
            pl.BlockSpec((1, hpg, 1, 1), lambda i, j: (i, j, 0, 0)),
            pl.BlockSpec((1, l, hpg * d), lambda i, j: (i, 0, j)),
            pl.BlockSpec((1, l, hpg * d), lambda i, j: (i, 0, j)),
            pl.BlockSpec((1, l, hpg * d), lambda i, j: (i, 0, j)),
        ],
        out_specs=pl.BlockSpec((1, l, hpg * d), lambda i, j: (i, 0, j)),
        out_shape=jax.ShapeDtypeStruct((b, l, hd), jnp.float32),
    )(m4, t4, q2, k2, v2)
    return out.reshape(b, l, h, d)
